# baseline (speedup 1.0000x reference)
import functools

import jax
import jax.numpy as jnp
from jax.experimental import pallas as pl

N = 10000
E = 160000
D = 256
ROW_BLK = 400


def _t1_body(x_ref, wc_ref, bc_ref, w1a_ref, b1_ref, w1b_ref,
             h_ref, a_ref, bm_ref):
    h = jax.nn.relu(
        jnp.dot(x_ref[...], wc_ref[...], preferred_element_type=jnp.float32)
        + bc_ref[...][None, :])
    h_ref[...] = h
    a_ref[...] = jnp.dot(h, w1a_ref[...], preferred_element_type=jnp.float32) + b1_ref[...][None, :]
    bm_ref[...] = jnp.dot(h, w1b_ref[...], preferred_element_type=jnp.float32)


def _t1(x, wc, bc, w1a, b1, w1b):
    grid = (N // ROW_BLK,)
    blk = pl.BlockSpec((ROW_BLK, D), lambda i: (i, 0))
    full = pl.BlockSpec((D, D), lambda i: (0, 0))
    vec = pl.BlockSpec((D,), lambda i: (0,))
    return pl.pallas_call(
        _t1_body,
        grid=grid,
        in_specs=[blk, full, vec, full, vec, full],
        out_specs=[blk, blk, blk],
        out_shape=[jax.ShapeDtypeStruct((N, D), jnp.float32)] * 3,
    )(x, wc, bc, w1a, b1, w1b)


def _t2_body(agg_ref, w_ref, b_ref, h_ref):
    h_ref[...] = jax.nn.relu(
        jnp.dot(agg_ref[...], w_ref[...], preferred_element_type=jnp.float32)
        + b_ref[...][None, :])


def _t2(agg, w, b):
    grid = (N // ROW_BLK,)
    blk = pl.BlockSpec((ROW_BLK, D), lambda i: (i, 0))
    return pl.pallas_call(
        _t2_body,
        grid=grid,
        in_specs=[blk, pl.BlockSpec((D, D), lambda i: (0, 0)),
                  pl.BlockSpec((D,), lambda i: (0,))],
        out_specs=blk,
        out_shape=jax.ShapeDtypeStruct((N, D), jnp.float32),
    )(agg, w, b)


def kernel(node_feats, edge_index, W_ctx, b_ctx, W_imp1, b_imp1, W_imp2,
           b_imp2, mask_temp, W_l0, b_l0, W_l1, b_l1):
    src = edge_index[0]
    dst = edge_index[1]
    w1a = W_imp1[:D, :]
    w1b = W_imp1[D:, :]
    h, A, B = _t1(node_feats, W_ctx, b_ctx, w1a, b_imp1, w1b)

    hid = jax.nn.relu(jnp.take(A, src, axis=0) + jnp.take(B, dst, axis=0))
    logit = hid @ W_imp2[:, 0] + b_imp2[0]
    imp = jax.nn.sigmoid(logit)
    masks = jax.nn.sigmoid((imp - 0.5) * jnp.exp(mask_temp))

    hcur = h
    for w, b in ((W_l0, b_l0), (W_l1, b_l1)):
        msg = jnp.take(hcur, src, axis=0) * masks[:, None]
        agg = jax.ops.segment_sum(msg, dst, num_segments=N)
        hcur = _t2(agg, w, b)
    sparsity = jnp.mean((masks < 0.5).astype(jnp.float32))
    return hcur, masks, sparsity
